# baseline (speedup 1.0000x reference)
import jax
import jax.numpy as jnp
from jax import lax
from jax.experimental import pallas as pl
from jax.experimental.pallas import tpu as pltpu
from jax.experimental.pallas import tpu_sc as plsc

N = 10000
E = 320000
D = 128

NC = 2
NS = 16
CHUNK = 80
EDGES_PER_TILE = E // (NC * NS)
NCHUNK = EDGES_PER_TILE // CHUNK

BLK = 2000


def _pre_body(x_ref, r_ref, w1_ref, w2_ref, w3_ref, b2_ref, b3_ref,
              sup_ref, trans_ref, gate1_ref):
    x = x_ref[...]
    sup_ref[...] = jnp.dot(x, w1_ref[...], preferred_element_type=jnp.float32)
    trans_ref[...] = jax.nn.sigmoid(
        jnp.dot(r_ref[...], w2_ref[...], preferred_element_type=jnp.float32)
        + b2_ref[...])
    gate1_ref[...] = (
        jnp.dot(x, w3_ref[...], preferred_element_type=jnp.float32)
        + b3_ref[...])


def _pre(x, r, w1, w2, w3, b2, b3):
    row = pl.BlockSpec((BLK, D), lambda i: (i, 0))
    mat = pl.BlockSpec((D, D), lambda i: (0, 0))
    vec = pl.BlockSpec((1, D), lambda i: (0, 0))
    out = jax.ShapeDtypeStruct((N, D), jnp.float32)
    return pl.pallas_call(
        _pre_body,
        grid=(N // BLK,),
        in_specs=[row, row, mat, mat, mat, vec, vec],
        out_specs=[row, row, row],
        out_shape=[out, out, out],
    )(x, r, w1, w2, w3, b2.reshape(1, D), b3.reshape(1, D))


def _agg_body(sup_hbm, src_hbm, dst_hbm, zeros_hbm, out_hbm,
              src_idx, dst_idx, rows0, rows1, agg_sh,
              semg0, semg1, sems0, sems1):
    c = lax.axis_index("c")
    s = lax.axis_index("s")
    base = pl.multiple_of((c * NS + s) * EDGES_PER_TILE, 8)

    pltpu.sync_copy(src_hbm.at[pl.ds(base, EDGES_PER_TILE)], src_idx)
    pltpu.sync_copy(dst_hbm.at[c, s], dst_idx)

    def issue_g(j, rows, sem):
        pltpu.async_copy(sup_hbm.at[src_idx.at[pl.ds(j * CHUNK, CHUNK)]],
                         rows, sem)

    def wait_g(j, rows, sem):
        pltpu.make_async_copy(
            sup_hbm.at[src_idx.at[pl.ds(j * CHUNK, CHUNK)]], rows, sem).wait()

    def issue_s(j, rows, sem):
        pltpu.async_copy(rows, agg_sh.at[dst_idx.at[j]], sem, add=True)

    def wait_s(j, rows, sem):
        pltpu.make_async_copy(rows, agg_sh.at[dst_idx.at[j]], sem).wait()

    @pl.when(s == 0)
    def _():
        pltpu.sync_copy(zeros_hbm, agg_sh)
    plsc.subcore_barrier()

    issue_g(0, rows0, semg0)
    wait_g(0, rows0, semg0)
    issue_s(0, rows0, sems0)
    issue_g(1, rows1, semg1)

    def body(i, carry):
        j = 2 * i + 1
        wait_g(j, rows1, semg1)
        issue_s(j, rows1, sems1)
        wait_s(j - 1, rows0, sems0)
        issue_g(j + 1, rows0, semg0)
        wait_g(j + 1, rows0, semg0)
        issue_s(j + 1, rows0, sems0)
        wait_s(j, rows1, sems1)
        issue_g(j + 2, rows1, semg1)
        return carry

    lax.fori_loop(0, (NCHUNK - 3) // 2, body, 0)
    j = NCHUNK - 2
    wait_g(j, rows1, semg1)
    issue_s(j, rows1, sems1)
    wait_s(j - 1, rows0, sems0)
    issue_g(j + 1, rows0, semg0)
    wait_g(j + 1, rows0, semg0)
    issue_s(j + 1, rows0, sems0)
    wait_s(j, rows1, sems1)
    wait_s(j + 1, rows0, sems0)

    plsc.subcore_barrier()
    rbase = pl.multiple_of(s * 624, 8)

    @pl.when(s < NS - 1)
    def _():
        pltpu.sync_copy(agg_sh.at[pl.ds(rbase, 624)],
                        out_hbm.at[c, pl.ds(rbase, 624)])

    @pl.when(s == NS - 1)
    def _():
        pltpu.sync_copy(agg_sh.at[pl.ds((NS - 1) * 624, 640)],
                        out_hbm.at[c, pl.ds((NS - 1) * 624, 640)])


def _sc_agg(sup, src, dst, zeros):
    mesh = plsc.VectorSubcoreMesh(core_axis_name="c", subcore_axis_name="s")
    f = pl.kernel(
        _agg_body,
        out_type=jax.ShapeDtypeStruct((NC, N, D), jnp.float32),
        mesh=mesh,
        scratch_types=[
            pltpu.VMEM((EDGES_PER_TILE,), jnp.int32),
            pltpu.VMEM((NCHUNK, CHUNK), jnp.int32),
            pltpu.VMEM((CHUNK, D), jnp.float32),
            pltpu.VMEM((CHUNK, D), jnp.float32),
            pltpu.VMEM_SHARED((N, D), jnp.float32),
            pltpu.SemaphoreType.DMA,
            pltpu.SemaphoreType.DMA,
            pltpu.SemaphoreType.DMA,
            pltpu.SemaphoreType.DMA,
        ],
    )
    return f(sup, src, dst, zeros)


def _post_body(p0_ref, p1_ref, sup_ref, gate1_ref, trans_ref, w4_ref,
               b1_ref, b4_ref, eps_ref, o1_ref, o2_ref):
    eps = eps_ref[0]
    out = p0_ref[...] + p1_ref[...] + eps * sup_ref[...] + b1_ref[...]
    out = jnp.maximum(out, 0.0)
    gate2 = (jnp.dot(out, w4_ref[...], preferred_element_type=jnp.float32)
             + b4_ref[...])
    gate = jax.nn.sigmoid(gate1_ref[...] + gate2)
    t = trans_ref[...]
    o1_ref[...] = out + gate * (t - out)
    o2_ref[...] = t + gate * (out - t)


def _post(p0, p1, sup, gate1, trans, w4, b1, b4, eps):
    row = pl.BlockSpec((BLK, D), lambda i: (i, 0))
    mat = pl.BlockSpec((D, D), lambda i: (0, 0))
    vec = pl.BlockSpec((1, D), lambda i: (0, 0))
    sca = pl.BlockSpec(memory_space=pltpu.SMEM)
    out = jax.ShapeDtypeStruct((N, D), jnp.float32)
    return pl.pallas_call(
        _post_body,
        grid=(N // BLK,),
        in_specs=[row, row, row, row, row, mat, vec, vec, sca],
        out_specs=[row, row],
        out_shape=[out, out],
    )(p0, p1, sup, gate1, trans, w4, b1.reshape(1, D), b4.reshape(1, D), eps)


def kernel(input, res_input, edge_index, w1, w2, w3, w4, epsilo, b1, b2, b3, b4):
    src = edge_index[0].astype(jnp.int32)
    dst = edge_index[1].astype(jnp.int32).reshape(NC, NS, NCHUNK, CHUNK)
    zeros = jnp.zeros((N, D), jnp.float32)

    support, trans, gate1 = _pre(input, res_input, w1, w2, w3, b2, b3)
    partials = _sc_agg(support, src, dst, zeros)
    return _post(partials[0], partials[1], support, gate1, trans,
                 w4, b1, b4, epsilo)


# speedup vs baseline: 1.2020x; 1.2020x over previous
import jax
import jax.numpy as jnp
from jax import lax
from jax.experimental import pallas as pl
from jax.experimental.pallas import tpu as pltpu
from jax.experimental.pallas import tpu_sc as plsc

N = 10000
E = 320000
D = 128

NC = 2
NS = 16
CHUNK = 80
EDGES_PER_TILE = E // (NC * NS)
NCHUNK = EDGES_PER_TILE // CHUNK

BLK = 2000


def _pre_body(x_ref, r_ref, w1_ref, w2_ref, w3_ref, b2_ref, b3_ref,
              sup_ref, trans_ref, gate1_ref):
    x = x_ref[...]
    sup_ref[...] = jnp.dot(x, w1_ref[...], preferred_element_type=jnp.float32)
    trans_ref[...] = jax.nn.sigmoid(
        jnp.dot(r_ref[...], w2_ref[...], preferred_element_type=jnp.float32)
        + b2_ref[...])
    gate1_ref[...] = (
        jnp.dot(x, w3_ref[...], preferred_element_type=jnp.float32)
        + b3_ref[...])


def _pre(x, r, w1, w2, w3, b2, b3):
    row = pl.BlockSpec((BLK, D), lambda i: (i, 0))
    mat = pl.BlockSpec((D, D), lambda i: (0, 0))
    vec = pl.BlockSpec((1, D), lambda i: (0, 0))
    out = jax.ShapeDtypeStruct((N, D), jnp.float32)
    return pl.pallas_call(
        _pre_body,
        grid=(N // BLK,),
        in_specs=[row, row, mat, mat, mat, vec, vec],
        out_specs=[row, row, row],
        out_shape=[out, out, out],
    )(x, r, w1, w2, w3, b2.reshape(1, D), b3.reshape(1, D))


def _agg_body(sup_hbm, src_hbm, dst_hbm, zeros_hbm, out_hbm,
              src_idx, dst_idx, rows0, rows1, agg_sh,
              semg0, semg1, sems0, sems1):
    c = lax.axis_index("c")
    s = lax.axis_index("s")
    base = pl.multiple_of((c * NS + s) * EDGES_PER_TILE, 8)

    pltpu.sync_copy(src_hbm.at[pl.ds(base, EDGES_PER_TILE)], src_idx)
    pltpu.sync_copy(dst_hbm.at[c, s], dst_idx)

    def issue_g(j, rows, sem):
        pltpu.async_copy(sup_hbm.at[src_idx.at[pl.ds(j * CHUNK, CHUNK)]],
                         rows, sem)

    def wait_g(j, rows, sem):
        pltpu.make_async_copy(
            sup_hbm.at[src_idx.at[pl.ds(j * CHUNK, CHUNK)]], rows, sem).wait()

    def issue_s(j, rows, sem):
        pltpu.async_copy(rows, agg_sh.at[dst_idx.at[j]], sem, add=True)

    def wait_s(j, rows, sem):
        pltpu.make_async_copy(rows, agg_sh.at[dst_idx.at[j]], sem).wait()

    @pl.when(s == 0)
    def _():
        pltpu.sync_copy(zeros_hbm, agg_sh)
    plsc.subcore_barrier()

    issue_g(0, rows0, semg0)
    issue_g(1, rows1, semg1)
    wait_g(0, rows0, semg0)
    issue_s(0, rows0, sems0)

    def body(i, carry):
        j = 2 * i + 1
        wait_s(j - 1, rows0, sems0)
        issue_g(j + 1, rows0, semg0)
        wait_g(j, rows1, semg1)
        issue_s(j, rows1, sems1)
        wait_s(j, rows1, sems1)
        issue_g(j + 2, rows1, semg1)
        wait_g(j + 1, rows0, semg0)
        issue_s(j + 1, rows0, sems0)
        return carry

    lax.fori_loop(0, (NCHUNK - 3) // 2, body, 0)
    j = NCHUNK - 2
    wait_s(j - 1, rows0, sems0)
    issue_g(j + 1, rows0, semg0)
    wait_g(j, rows1, semg1)
    issue_s(j, rows1, sems1)
    wait_s(j, rows1, sems1)
    wait_g(j + 1, rows0, semg0)
    issue_s(j + 1, rows0, sems0)
    wait_s(j + 1, rows0, sems0)

    plsc.subcore_barrier()
    rbase = pl.multiple_of(s * 624, 8)

    @pl.when(s < NS - 1)
    def _():
        pltpu.sync_copy(agg_sh.at[pl.ds(rbase, 624)],
                        out_hbm.at[c, pl.ds(rbase, 624)])

    @pl.when(s == NS - 1)
    def _():
        pltpu.sync_copy(agg_sh.at[pl.ds((NS - 1) * 624, 640)],
                        out_hbm.at[c, pl.ds((NS - 1) * 624, 640)])


def _sc_agg(sup, src, dst, zeros):
    mesh = plsc.VectorSubcoreMesh(core_axis_name="c", subcore_axis_name="s")
    f = pl.kernel(
        _agg_body,
        out_type=jax.ShapeDtypeStruct((NC, N, D), jnp.float32),
        mesh=mesh,
        scratch_types=[
            pltpu.VMEM((EDGES_PER_TILE,), jnp.int32),
            pltpu.VMEM((NCHUNK, CHUNK), jnp.int32),
            pltpu.VMEM((CHUNK, D), jnp.float32),
            pltpu.VMEM((CHUNK, D), jnp.float32),
            pltpu.VMEM_SHARED((N, D), jnp.float32),
            pltpu.SemaphoreType.DMA,
            pltpu.SemaphoreType.DMA,
            pltpu.SemaphoreType.DMA,
            pltpu.SemaphoreType.DMA,
        ],
    )
    return f(sup, src, dst, zeros)


def _post_body(p0_ref, p1_ref, sup_ref, gate1_ref, trans_ref, w4_ref,
               b1_ref, b4_ref, eps_ref, o1_ref, o2_ref):
    eps = eps_ref[0]
    out = p0_ref[...] + p1_ref[...] + eps * sup_ref[...] + b1_ref[...]
    out = jnp.maximum(out, 0.0)
    gate2 = (jnp.dot(out, w4_ref[...], preferred_element_type=jnp.float32)
             + b4_ref[...])
    gate = jax.nn.sigmoid(gate1_ref[...] + gate2)
    t = trans_ref[...]
    o1_ref[...] = out + gate * (t - out)
    o2_ref[...] = t + gate * (out - t)


def _post(p0, p1, sup, gate1, trans, w4, b1, b4, eps):
    row = pl.BlockSpec((BLK, D), lambda i: (i, 0))
    mat = pl.BlockSpec((D, D), lambda i: (0, 0))
    vec = pl.BlockSpec((1, D), lambda i: (0, 0))
    sca = pl.BlockSpec(memory_space=pltpu.SMEM)
    out = jax.ShapeDtypeStruct((N, D), jnp.float32)
    return pl.pallas_call(
        _post_body,
        grid=(N // BLK,),
        in_specs=[row, row, row, row, row, mat, vec, vec, sca],
        out_specs=[row, row],
        out_shape=[out, out],
    )(p0, p1, sup, gate1, trans, w4, b1.reshape(1, D), b4.reshape(1, D), eps)


def kernel(input, res_input, edge_index, w1, w2, w3, w4, epsilo, b1, b2, b3, b4):
    src = edge_index[0].astype(jnp.int32)
    dst = edge_index[1].astype(jnp.int32).reshape(NC, NS, NCHUNK, CHUNK)
    zeros = jnp.zeros((N, D), jnp.float32)

    support, trans, gate1 = _pre(input, res_input, w1, w2, w3, b2, b3)
    partials = _sc_agg(support, src, dst, zeros)
    return _post(partials[0], partials[1], support, gate1, trans,
                 w4, b1, b4, epsilo)


# speedup vs baseline: 1.2607x; 1.0488x over previous
import jax
import jax.numpy as jnp
from jax import lax
from jax.experimental import pallas as pl
from jax.experimental.pallas import tpu as pltpu
from jax.experimental.pallas import tpu_sc as plsc

N = 10000
E = 320000
D = 128

NC = 2
NS = 16
CHUNK = 80
EDGES_PER_TILE = E // (NC * NS)
NCHUNK = EDGES_PER_TILE // CHUNK

BLK = 2000


def _pre_sup_body(x_ref, w1_ref, sup_ref):
    sup_ref[...] = jnp.dot(x_ref[...], w1_ref[...],
                           preferred_element_type=jnp.float32)


def _pre_sup(x, w1):
    row = pl.BlockSpec((BLK, D), lambda i: (i, 0))
    mat = pl.BlockSpec((D, D), lambda i: (0, 0))
    return pl.pallas_call(
        _pre_sup_body,
        grid=(N // BLK,),
        in_specs=[row, mat],
        out_specs=row,
        out_shape=jax.ShapeDtypeStruct((N, D), jnp.float32),
    )(x, w1)


def _pre_rest_body(x_ref, r_ref, w2_ref, w3_ref, b2_ref, b3_ref,
                   trans_ref, gate1_ref):
    trans_ref[...] = jax.nn.sigmoid(
        jnp.dot(r_ref[...], w2_ref[...], preferred_element_type=jnp.float32)
        + b2_ref[...])
    gate1_ref[...] = (
        jnp.dot(x_ref[...], w3_ref[...], preferred_element_type=jnp.float32)
        + b3_ref[...])


def _pre_rest(x, r, w2, w3, b2, b3):
    row = pl.BlockSpec((BLK, D), lambda i: (i, 0))
    mat = pl.BlockSpec((D, D), lambda i: (0, 0))
    vec = pl.BlockSpec((1, D), lambda i: (0, 0))
    out = jax.ShapeDtypeStruct((N, D), jnp.float32)
    return pl.pallas_call(
        _pre_rest_body,
        grid=(N // BLK,),
        in_specs=[row, row, mat, mat, vec, vec],
        out_specs=[row, row],
        out_shape=[out, out],
    )(x, r, w2, w3, b2.reshape(1, D), b3.reshape(1, D))


def _agg_body(sup_hbm, src_hbm, dst_hbm, zeros_hbm, out_hbm,
              src_idx, dst_idx, rows0, rows1, agg_sh,
              semg0, semg1, sems0, sems1):
    c = lax.axis_index("c")
    s = lax.axis_index("s")
    base = pl.multiple_of((c * NS + s) * EDGES_PER_TILE, 8)

    pltpu.sync_copy(src_hbm.at[pl.ds(base, EDGES_PER_TILE)], src_idx)
    pltpu.sync_copy(dst_hbm.at[c, s], dst_idx)

    def issue_g(j, rows, sem):
        pltpu.async_copy(sup_hbm.at[src_idx.at[pl.ds(j * CHUNK, CHUNK)]],
                         rows, sem)

    def wait_g(j, rows, sem):
        pltpu.make_async_copy(
            sup_hbm.at[src_idx.at[pl.ds(j * CHUNK, CHUNK)]], rows, sem).wait()

    def issue_s(j, rows, sem):
        pltpu.async_copy(rows, agg_sh.at[dst_idx.at[j]], sem, add=True)

    def wait_s(j, rows, sem):
        pltpu.make_async_copy(rows, agg_sh.at[dst_idx.at[j]], sem).wait()

    @pl.when(s == 0)
    def _():
        pltpu.sync_copy(zeros_hbm, agg_sh)
    plsc.subcore_barrier()

    issue_g(0, rows0, semg0)
    issue_g(1, rows1, semg1)
    wait_g(0, rows0, semg0)
    issue_s(0, rows0, sems0)

    def body(i, carry):
        j = 2 * i + 1
        wait_s(j - 1, rows0, sems0)
        issue_g(j + 1, rows0, semg0)
        wait_g(j, rows1, semg1)
        issue_s(j, rows1, sems1)
        wait_s(j, rows1, sems1)
        issue_g(j + 2, rows1, semg1)
        wait_g(j + 1, rows0, semg0)
        issue_s(j + 1, rows0, sems0)
        return carry

    lax.fori_loop(0, (NCHUNK - 3) // 2, body, 0)
    j = NCHUNK - 2
    wait_s(j - 1, rows0, sems0)
    issue_g(j + 1, rows0, semg0)
    wait_g(j, rows1, semg1)
    issue_s(j, rows1, sems1)
    wait_s(j, rows1, sems1)
    wait_g(j + 1, rows0, semg0)
    issue_s(j + 1, rows0, sems0)
    wait_s(j + 1, rows0, sems0)

    plsc.subcore_barrier()
    rbase = pl.multiple_of(s * 624, 8)

    @pl.when(s < NS - 1)
    def _():
        pltpu.sync_copy(agg_sh.at[pl.ds(rbase, 624)],
                        out_hbm.at[c, pl.ds(rbase, 624)])

    @pl.when(s == NS - 1)
    def _():
        pltpu.sync_copy(agg_sh.at[pl.ds((NS - 1) * 624, 640)],
                        out_hbm.at[c, pl.ds((NS - 1) * 624, 640)])


def _sc_agg(sup, src, dst, zeros):
    mesh = plsc.VectorSubcoreMesh(core_axis_name="c", subcore_axis_name="s")
    f = pl.kernel(
        _agg_body,
        out_type=jax.ShapeDtypeStruct((NC, N, D), jnp.float32),
        mesh=mesh,
        cost_estimate=pl.CostEstimate(
            flops=2 * E * D,
            bytes_accessed=2 * E * D * 4 + 3 * N * D * 4,
            transcendentals=0),
        scratch_types=[
            pltpu.VMEM((EDGES_PER_TILE,), jnp.int32),
            pltpu.VMEM((NCHUNK, CHUNK), jnp.int32),
            pltpu.VMEM((CHUNK, D), jnp.float32),
            pltpu.VMEM((CHUNK, D), jnp.float32),
            pltpu.VMEM_SHARED((N, D), jnp.float32),
            pltpu.SemaphoreType.DMA,
            pltpu.SemaphoreType.DMA,
            pltpu.SemaphoreType.DMA,
            pltpu.SemaphoreType.DMA,
        ],
    )
    return f(sup, src, dst, zeros)


def _post_body(p0_ref, p1_ref, sup_ref, gate1_ref, trans_ref, w4_ref,
               b1_ref, b4_ref, eps_ref, o1_ref, o2_ref):
    eps = eps_ref[0]
    out = (p0_ref[0] + p1_ref[0]) + eps * sup_ref[...] + b1_ref[...]
    out = jnp.maximum(out, 0.0)
    gate2 = (jnp.dot(out, w4_ref[...], preferred_element_type=jnp.float32)
             + b4_ref[...])
    gate = jax.nn.sigmoid(gate1_ref[...] + gate2)
    t = trans_ref[...]
    o1_ref[...] = out + gate * (t - out)
    o2_ref[...] = t + gate * (out - t)


def _post(partials, sup, gate1, trans, w4, b1, b4, eps):
    row = pl.BlockSpec((BLK, D), lambda i: (i, 0))
    par0 = pl.BlockSpec((1, BLK, D), lambda i: (0, i, 0))
    par1 = pl.BlockSpec((1, BLK, D), lambda i: (1, i, 0))
    mat = pl.BlockSpec((D, D), lambda i: (0, 0))
    vec = pl.BlockSpec((1, D), lambda i: (0, 0))
    sca = pl.BlockSpec(memory_space=pltpu.SMEM)
    out = jax.ShapeDtypeStruct((N, D), jnp.float32)
    return pl.pallas_call(
        _post_body,
        grid=(N // BLK,),
        in_specs=[par0, par1, row, row, row, mat, vec, vec, sca],
        out_specs=[row, row],
        out_shape=[out, out],
    )(partials, partials, sup, gate1, trans, w4,
      b1.reshape(1, D), b4.reshape(1, D), eps)


def kernel(input, res_input, edge_index, w1, w2, w3, w4, epsilo, b1, b2, b3, b4):
    src = edge_index[0].astype(jnp.int32)
    dst = edge_index[1].astype(jnp.int32).reshape(NC, NS, NCHUNK, CHUNK)
    zeros = jnp.zeros((N, D), jnp.float32)

    support = _pre_sup(input, w1)
    partials = _sc_agg(support, src, dst, zeros)
    trans, gate1 = _pre_rest(input, res_input, w2, w3, b2, b3)
    return _post(partials, support, gate1, trans, w4, b1, b4, epsilo)


# speedup vs baseline: 1.4379x; 1.1406x over previous
import jax
import jax.numpy as jnp
from jax import lax
from jax.experimental import pallas as pl
from jax.experimental.pallas import tpu as pltpu
from jax.experimental.pallas import tpu_sc as plsc

N = 10000
E = 320000
D = 128

NC = 2
NS = 16
CHUNK = 80
EDGES_PER_TILE = E // (NC * NS)
NCHUNK = EDGES_PER_TILE // CHUNK

BLK = 2000


def _pre_sup_body(x_ref, w1_ref, sup_ref):
    sup_ref[...] = jnp.dot(x_ref[...], w1_ref[...],
                           preferred_element_type=jnp.float32)


def _pre_sup(x, w1):
    row = pl.BlockSpec((BLK, D), lambda i: (i, 0))
    mat = pl.BlockSpec((D, D), lambda i: (0, 0))
    return pl.pallas_call(
        _pre_sup_body,
        grid=(N // BLK,),
        in_specs=[row, mat],
        out_specs=row,
        out_shape=jax.ShapeDtypeStruct((N, D), jnp.float32),
    )(x, w1)


def _pre_rest_body(x_ref, r_ref, w2_ref, w3_ref, b2_ref, b3_ref,
                   trans_ref, gate1_ref):
    trans_ref[...] = jax.nn.sigmoid(
        jnp.dot(r_ref[...], w2_ref[...], preferred_element_type=jnp.float32)
        + b2_ref[...])
    gate1_ref[...] = (
        jnp.dot(x_ref[...], w3_ref[...], preferred_element_type=jnp.float32)
        + b3_ref[...])


def _pre_rest(x, r, w2, w3, b2, b3):
    row = pl.BlockSpec((BLK, D), lambda i: (i, 0))
    mat = pl.BlockSpec((D, D), lambda i: (0, 0))
    vec = pl.BlockSpec((1, D), lambda i: (0, 0))
    out = jax.ShapeDtypeStruct((N, D), jnp.float32)
    return pl.pallas_call(
        _pre_rest_body,
        grid=(N // BLK,),
        in_specs=[row, row, mat, mat, vec, vec],
        out_specs=[row, row],
        out_shape=[out, out],
    )(x, r, w2, w3, b2.reshape(1, D), b3.reshape(1, D))


def _agg_body(sup_hbm, src_hbm, dst_hbm, zeros_hbm, out_hbm,
              src_idx, dst_idx, rows0, rows1, rows2, agg_sh,
              semg0, semg1, semg2, sems0, sems1, sems2):
    c = lax.axis_index("c")
    s = lax.axis_index("s")
    base = pl.multiple_of((c * NS + s) * EDGES_PER_TILE, 8)

    pltpu.sync_copy(src_hbm.at[pl.ds(base, EDGES_PER_TILE)], src_idx)
    pltpu.sync_copy(dst_hbm.at[pl.ds(base, EDGES_PER_TILE)], dst_idx)

    def issue_g(j, rows, sem):
        pltpu.async_copy(sup_hbm.at[src_idx.at[pl.ds(j * CHUNK, CHUNK)]],
                         rows, sem)

    def wait_g(j, rows, sem):
        pltpu.make_async_copy(
            sup_hbm.at[src_idx.at[pl.ds(j * CHUNK, CHUNK)]], rows, sem).wait()

    def issue_s(j, rows, sem):
        pltpu.async_copy(
            rows, agg_sh.at[dst_idx.at[pl.ds(j * CHUNK, CHUNK)]], sem,
            add=True)

    def wait_s(j, rows, sem):
        pltpu.make_async_copy(
            rows, agg_sh.at[dst_idx.at[pl.ds(j * CHUNK, CHUNK)]], sem).wait()

    @pl.when(s == 0)
    def _():
        pltpu.sync_copy(zeros_hbm, agg_sh)
    plsc.subcore_barrier()

    bufs = ((rows0, semg0, sems0), (rows1, semg1, sems1),
            (rows2, semg2, sems2))

    def step(j, b, guard_next=False):
        rows, semg, sems = bufs[b]
        prows, _, psems = bufs[(b + 2) % 3]
        wait_g(j, rows, semg)
        issue_s(j, rows, sems)
        wait_s(j - 1, prows, psems)
        if guard_next:
            @pl.when(j + 2 < NCHUNK)
            def _():
                issue_g(j + 2, prows, bufs[(b + 2) % 3][1])
        else:
            issue_g(j + 2, prows, bufs[(b + 2) % 3][1])

    issue_g(0, rows0, semg0)
    issue_g(1, rows1, semg1)
    wait_g(0, rows0, semg0)
    issue_s(0, rows0, sems0)
    issue_g(2, rows2, semg2)
    wait_g(1, rows1, semg1)
    issue_s(1, rows1, sems1)
    wait_s(0, rows0, sems0)
    issue_g(3, rows0, semg0)

    def body(i, carry):
        j = 3 * i + 2
        step(j, 2)
        step(j + 1, 0, guard_next=True)
        step(j + 2, 1, guard_next=True)
        return carry

    lax.fori_loop(0, (NCHUNK - 2) // 3, body, 0)
    wait_s(NCHUNK - 1, rows1, sems1)

    plsc.subcore_barrier()
    rbase = pl.multiple_of(s * 624, 8)

    @pl.when(s < NS - 1)
    def _():
        pltpu.sync_copy(agg_sh.at[pl.ds(rbase, 624)],
                        out_hbm.at[c, pl.ds(rbase, 624)])

    @pl.when(s == NS - 1)
    def _():
        pltpu.sync_copy(agg_sh.at[pl.ds((NS - 1) * 624, 640)],
                        out_hbm.at[c, pl.ds((NS - 1) * 624, 640)])


def _sc_agg(sup, src, dst, zeros):
    mesh = plsc.VectorSubcoreMesh(core_axis_name="c", subcore_axis_name="s")
    f = pl.kernel(
        _agg_body,
        out_type=jax.ShapeDtypeStruct((NC, N, D), jnp.float32),
        mesh=mesh,
        cost_estimate=pl.CostEstimate(
            flops=2 * E * D,
            bytes_accessed=2 * E * D * 4 + 3 * N * D * 4,
            transcendentals=0),
        scratch_types=[
            pltpu.VMEM((EDGES_PER_TILE,), jnp.int32),
            pltpu.VMEM((EDGES_PER_TILE,), jnp.int32),
            pltpu.VMEM((CHUNK, D), jnp.float32),
            pltpu.VMEM((CHUNK, D), jnp.float32),
            pltpu.VMEM((CHUNK, D), jnp.float32),
            pltpu.VMEM_SHARED((N, D), jnp.float32),
            pltpu.SemaphoreType.DMA,
            pltpu.SemaphoreType.DMA,
            pltpu.SemaphoreType.DMA,
            pltpu.SemaphoreType.DMA,
            pltpu.SemaphoreType.DMA,
            pltpu.SemaphoreType.DMA,
        ],
    )
    return f(sup, src, dst, zeros)


def _post_body(p0_ref, p1_ref, sup_ref, gate1_ref, trans_ref, w4_ref,
               b1_ref, b4_ref, eps_ref, o1_ref, o2_ref):
    eps = eps_ref[0]
    out = (p0_ref[0] + p1_ref[0]) + eps * sup_ref[...] + b1_ref[...]
    out = jnp.maximum(out, 0.0)
    gate2 = (jnp.dot(out, w4_ref[...], preferred_element_type=jnp.float32)
             + b4_ref[...])
    gate = jax.nn.sigmoid(gate1_ref[...] + gate2)
    t = trans_ref[...]
    o1_ref[...] = out + gate * (t - out)
    o2_ref[...] = t + gate * (out - t)


def _post(partials, sup, gate1, trans, w4, b1, b4, eps):
    row = pl.BlockSpec((BLK, D), lambda i: (i, 0))
    par0 = pl.BlockSpec((1, BLK, D), lambda i: (0, i, 0))
    par1 = pl.BlockSpec((1, BLK, D), lambda i: (1, i, 0))
    mat = pl.BlockSpec((D, D), lambda i: (0, 0))
    vec = pl.BlockSpec((1, D), lambda i: (0, 0))
    sca = pl.BlockSpec(memory_space=pltpu.SMEM)
    out = jax.ShapeDtypeStruct((N, D), jnp.float32)
    return pl.pallas_call(
        _post_body,
        grid=(N // BLK,),
        in_specs=[par0, par1, row, row, row, mat, vec, vec, sca],
        out_specs=[row, row],
        out_shape=[out, out],
    )(partials, partials, sup, gate1, trans, w4,
      b1.reshape(1, D), b4.reshape(1, D), eps)


def kernel(input, res_input, edge_index, w1, w2, w3, w4, epsilo, b1, b2, b3, b4):
    src = edge_index[0].astype(jnp.int32)
    dst = edge_index[1].astype(jnp.int32)
    zeros = jnp.zeros((N, D), jnp.float32)

    support = _pre_sup(input, w1)
    partials = _sc_agg(support, src, dst, zeros)
    trans, gate1 = _pre_rest(input, res_input, w2, w3, b2, b3)
    return _post(partials, support, gate1, trans, w4, b1, b4, epsilo)


# speedup vs baseline: 1.4596x; 1.0151x over previous
import jax
import jax.numpy as jnp
from jax import lax
from jax.experimental import pallas as pl
from jax.experimental.pallas import tpu as pltpu
from jax.experimental.pallas import tpu_sc as plsc

N = 10000
E = 320000
D = 128

NC = 2
NS = 16
CHUNK = 80
EDGES_PER_TILE = E // (NC * NS)
NCHUNK = EDGES_PER_TILE // CHUNK

BLK = 2000


def _pre_sup_body(x_ref, w1_ref, sup_ref):
    sup_ref[...] = jnp.dot(x_ref[...], w1_ref[...],
                           preferred_element_type=jnp.float32)


def _pre_sup(x, w1):
    row = pl.BlockSpec((BLK, D), lambda i: (i, 0))
    mat = pl.BlockSpec((D, D), lambda i: (0, 0))
    return pl.pallas_call(
        _pre_sup_body,
        grid=(N // BLK,),
        in_specs=[row, mat],
        out_specs=row,
        out_shape=jax.ShapeDtypeStruct((N, D), jnp.float32),
    )(x, w1)


def _agg_body(sup_hbm, src_hbm, dst_hbm, zeros_hbm, out_hbm,
              src_idx, dst_idx, rows0, rows1, rows2, agg_sh,
              semg0, semg1, semg2, sems0, sems1, sems2):
    c = lax.axis_index("c")
    s = lax.axis_index("s")
    base = pl.multiple_of((c * NS + s) * EDGES_PER_TILE, 8)

    pltpu.sync_copy(src_hbm.at[pl.ds(base, EDGES_PER_TILE)], src_idx)
    pltpu.sync_copy(dst_hbm.at[pl.ds(base, EDGES_PER_TILE)], dst_idx)

    def issue_g(j, rows, sem):
        pltpu.async_copy(sup_hbm.at[src_idx.at[pl.ds(j * CHUNK, CHUNK)]],
                         rows, sem)

    def wait_g(j, rows, sem):
        pltpu.make_async_copy(
            sup_hbm.at[src_idx.at[pl.ds(j * CHUNK, CHUNK)]], rows, sem).wait()

    def issue_s(j, rows, sem):
        pltpu.async_copy(
            rows, agg_sh.at[dst_idx.at[pl.ds(j * CHUNK, CHUNK)]], sem,
            add=True)

    def wait_s(j, rows, sem):
        pltpu.make_async_copy(
            rows, agg_sh.at[dst_idx.at[pl.ds(j * CHUNK, CHUNK)]], sem).wait()

    bufs = ((rows0, semg0, sems0), (rows1, semg1, sems1),
            (rows2, semg2, sems2))

    def step(j, b, guard_next=False):
        rows, semg, sems = bufs[b]
        prows, _, psems = bufs[(b + 2) % 3]
        wait_g(j, rows, semg)
        issue_s(j, rows, sems)
        wait_s(j - 1, prows, psems)
        if guard_next:
            @pl.when(j + 2 < NCHUNK)
            def _():
                issue_g(j + 2, prows, bufs[(b + 2) % 3][1])
        else:
            issue_g(j + 2, prows, bufs[(b + 2) % 3][1])

    issue_g(0, rows0, semg0)
    issue_g(1, rows1, semg1)

    @pl.when(s == 0)
    def _():
        pltpu.sync_copy(zeros_hbm, agg_sh)
    plsc.subcore_barrier()

    wait_g(0, rows0, semg0)
    issue_s(0, rows0, sems0)
    issue_g(2, rows2, semg2)
    wait_g(1, rows1, semg1)
    issue_s(1, rows1, sems1)
    wait_s(0, rows0, sems0)
    issue_g(3, rows0, semg0)

    def body(i, carry):
        j = 3 * i + 2
        step(j, 2)
        step(j + 1, 0, guard_next=True)
        step(j + 2, 1, guard_next=True)
        return carry

    lax.fori_loop(0, (NCHUNK - 2) // 3, body, 0)
    wait_s(NCHUNK - 1, rows1, sems1)

    plsc.subcore_barrier()
    rbase = pl.multiple_of(s * 624, 8)

    @pl.when(s < NS - 1)
    def _():
        pltpu.sync_copy(agg_sh.at[pl.ds(rbase, 624)],
                        out_hbm.at[c, pl.ds(rbase, 624)])

    @pl.when(s == NS - 1)
    def _():
        pltpu.sync_copy(agg_sh.at[pl.ds((NS - 1) * 624, 640)],
                        out_hbm.at[c, pl.ds((NS - 1) * 624, 640)])


def _sc_agg(sup, src, dst, zeros):
    mesh = plsc.VectorSubcoreMesh(core_axis_name="c", subcore_axis_name="s")
    f = pl.kernel(
        _agg_body,
        out_type=jax.ShapeDtypeStruct((NC, N, D), jnp.float32),
        mesh=mesh,
        cost_estimate=pl.CostEstimate(
            flops=2 * E * D,
            bytes_accessed=2 * E * D * 4 + 3 * N * D * 4,
            transcendentals=0),
        scratch_types=[
            pltpu.VMEM((EDGES_PER_TILE,), jnp.int32),
            pltpu.VMEM((EDGES_PER_TILE,), jnp.int32),
            pltpu.VMEM((CHUNK, D), jnp.float32),
            pltpu.VMEM((CHUNK, D), jnp.float32),
            pltpu.VMEM((CHUNK, D), jnp.float32),
            pltpu.VMEM_SHARED((N, D), jnp.float32),
            pltpu.SemaphoreType.DMA,
            pltpu.SemaphoreType.DMA,
            pltpu.SemaphoreType.DMA,
            pltpu.SemaphoreType.DMA,
            pltpu.SemaphoreType.DMA,
            pltpu.SemaphoreType.DMA,
        ],
    )
    return f(sup, src, dst, zeros)


def _post_body(p0_ref, p1_ref, sup_ref, x_ref, r_ref, w2_ref, w3_ref,
               w4_ref, b1_ref, b2_ref, b3_ref, b4_ref, eps_ref,
               o1_ref, o2_ref):
    eps = eps_ref[0]
    t = jax.nn.sigmoid(
        jnp.dot(r_ref[...], w2_ref[...], preferred_element_type=jnp.float32)
        + b2_ref[...])
    gate1 = (jnp.dot(x_ref[...], w3_ref[...],
                     preferred_element_type=jnp.float32) + b3_ref[...])
    out = (p0_ref[0] + p1_ref[0]) + eps * sup_ref[...] + b1_ref[...]
    out = jnp.maximum(out, 0.0)
    gate2 = (jnp.dot(out, w4_ref[...], preferred_element_type=jnp.float32)
             + b4_ref[...])
    gate = jax.nn.sigmoid(gate1 + gate2)
    o1_ref[...] = out + gate * (t - out)
    o2_ref[...] = t + gate * (out - t)


def _post(partials, sup, x, r, w2, w3, w4, b1, b2, b3, b4, eps):
    row = pl.BlockSpec((BLK, D), lambda i: (i, 0))
    par0 = pl.BlockSpec((1, BLK, D), lambda i: (0, i, 0))
    par1 = pl.BlockSpec((1, BLK, D), lambda i: (1, i, 0))
    mat = pl.BlockSpec((D, D), lambda i: (0, 0))
    vec = pl.BlockSpec((1, D), lambda i: (0, 0))
    sca = pl.BlockSpec(memory_space=pltpu.SMEM)
    out = jax.ShapeDtypeStruct((N, D), jnp.float32)
    return pl.pallas_call(
        _post_body,
        grid=(N // BLK,),
        in_specs=[par0, par1, row, row, row, mat, mat, mat,
                  vec, vec, vec, vec, sca],
        out_specs=[row, row],
        out_shape=[out, out],
    )(partials, partials, sup, x, r, w2, w3, w4,
      b1.reshape(1, D), b2.reshape(1, D), b3.reshape(1, D),
      b4.reshape(1, D), eps)


def kernel(input, res_input, edge_index, w1, w2, w3, w4, epsilo, b1, b2, b3, b4):
    src = edge_index[0].astype(jnp.int32)
    dst = edge_index[1].astype(jnp.int32)
    zeros = jnp.zeros((N, D), jnp.float32)

    support = _pre_sup(input, w1)
    partials = _sc_agg(support, src, dst, zeros)
    return _post(partials, support, input, res_input,
                 w2, w3, w4, b1, b2, b3, b4, epsilo)


# speedup vs baseline: 1.4785x; 1.0130x over previous
import jax
import jax.numpy as jnp
from jax import lax
from jax.experimental import pallas as pl
from jax.experimental.pallas import tpu as pltpu
from jax.experimental.pallas import tpu_sc as plsc

N = 10000
E = 320000
D = 128

NC = 2
NS = 16
CHUNK = 40
EDGES_PER_TILE = E // (NC * NS)
NCHUNK = EDGES_PER_TILE // CHUNK
NBUF = 5

BLK = 2000


def _pre_sup_body(x_ref, w1_ref, sup_ref):
    sup_ref[...] = jnp.dot(x_ref[...], w1_ref[...],
                           preferred_element_type=jnp.float32)


def _pre_sup(x, w1):
    row = pl.BlockSpec((BLK, D), lambda i: (i, 0))
    mat = pl.BlockSpec((D, D), lambda i: (0, 0))
    return pl.pallas_call(
        _pre_sup_body,
        grid=(N // BLK,),
        in_specs=[row, mat],
        out_specs=row,
        out_shape=jax.ShapeDtypeStruct((N, D), jnp.float32),
    )(x, w1)


def _agg_body(sup_hbm, src_hbm, dst_hbm, zeros_hbm, out_hbm,
              src_idx, dst_idx, rows0, rows1, rows2, rows3, rows4, agg_sh,
              semg0, semg1, semg2, semg3, semg4,
              sems0, sems1, sems2, sems3, sems4):
    c = lax.axis_index("c")
    s = lax.axis_index("s")
    base = pl.multiple_of((c * NS + s) * EDGES_PER_TILE, 8)

    pltpu.sync_copy(src_hbm.at[pl.ds(base, EDGES_PER_TILE)], src_idx)
    pltpu.sync_copy(dst_hbm.at[pl.ds(base, EDGES_PER_TILE)], dst_idx)

    def issue_g(j, rows, sem):
        pltpu.async_copy(sup_hbm.at[src_idx.at[pl.ds(j * CHUNK, CHUNK)]],
                         rows, sem)

    def wait_g(j, rows, sem):
        pltpu.make_async_copy(
            sup_hbm.at[src_idx.at[pl.ds(j * CHUNK, CHUNK)]], rows, sem).wait()

    def issue_s(j, rows, sem):
        pltpu.async_copy(
            rows, agg_sh.at[dst_idx.at[pl.ds(j * CHUNK, CHUNK)]], sem,
            add=True)

    def wait_s(j, rows, sem):
        pltpu.make_async_copy(
            rows, agg_sh.at[dst_idx.at[pl.ds(j * CHUNK, CHUNK)]], sem).wait()

    bufs = ((rows0, semg0, sems0), (rows1, semg1, sems1),
            (rows2, semg2, sems2), (rows3, semg3, sems3),
            (rows4, semg4, sems4))

    def step(j, b, issue_next=True, first=False):
        rows, semg, sems = bufs[b]
        prows, psemg, psems = bufs[(b + NBUF - 1) % NBUF]
        wait_g(j, rows, semg)
        issue_s(j, rows, sems)
        if not first:
            wait_s(j - 1, prows, psems)
        if issue_next:
            issue_g(j + NBUF - 1, prows, psemg)

    for k in range(NBUF - 1):
        issue_g(k, bufs[k][0], bufs[k][1])

    @pl.when(s == 0)
    def _():
        pltpu.sync_copy(zeros_hbm, agg_sh)
    plsc.subcore_barrier()

    step(0, 0, first=True)

    def body(i, carry):
        j0 = NBUF * i + 1
        for k in range(NBUF):
            step(j0 + k, (1 + k) % NBUF)
        return carry

    lax.fori_loop(0, (NCHUNK - NBUF) // NBUF, body, 0)
    for j in range(NCHUNK - NBUF + 1, NCHUNK):
        step(j, j % NBUF, issue_next=False)
    wait_s(NCHUNK - 1, bufs[(NCHUNK - 1) % NBUF][0],
           bufs[(NCHUNK - 1) % NBUF][2])

    plsc.subcore_barrier()
    rbase = pl.multiple_of(s * 624, 8)

    @pl.when(s < NS - 1)
    def _():
        pltpu.sync_copy(agg_sh.at[pl.ds(rbase, 624)],
                        out_hbm.at[c, pl.ds(rbase, 624)])

    @pl.when(s == NS - 1)
    def _():
        pltpu.sync_copy(agg_sh.at[pl.ds((NS - 1) * 624, 640)],
                        out_hbm.at[c, pl.ds((NS - 1) * 624, 640)])


def _sc_agg(sup, src, dst, zeros):
    mesh = plsc.VectorSubcoreMesh(core_axis_name="c", subcore_axis_name="s")
    f = pl.kernel(
        _agg_body,
        out_type=jax.ShapeDtypeStruct((NC, N, D), jnp.float32),
        mesh=mesh,
        cost_estimate=pl.CostEstimate(
            flops=2 * E * D,
            bytes_accessed=2 * E * D * 4 + 3 * N * D * 4,
            transcendentals=0),
        scratch_types=[
            pltpu.VMEM((EDGES_PER_TILE,), jnp.int32),
            pltpu.VMEM((EDGES_PER_TILE,), jnp.int32),
        ] + [pltpu.VMEM((CHUNK, D), jnp.float32) for _ in range(NBUF)] + [
            pltpu.VMEM_SHARED((N, D), jnp.float32),
        ] + [pltpu.SemaphoreType.DMA for _ in range(2 * NBUF)],
    )
    return f(sup, src, dst, zeros)


def _post_body(p0_ref, p1_ref, sup_ref, x_ref, r_ref, w2_ref, w3_ref,
               w4_ref, b1_ref, b2_ref, b3_ref, b4_ref, eps_ref,
               o1_ref, o2_ref):
    eps = eps_ref[0]
    t = jax.nn.sigmoid(
        jnp.dot(r_ref[...], w2_ref[...], preferred_element_type=jnp.float32)
        + b2_ref[...])
    gate1 = (jnp.dot(x_ref[...], w3_ref[...],
                     preferred_element_type=jnp.float32) + b3_ref[...])
    out = (p0_ref[0] + p1_ref[0]) + eps * sup_ref[...] + b1_ref[...]
    out = jnp.maximum(out, 0.0)
    gate2 = (jnp.dot(out, w4_ref[...], preferred_element_type=jnp.float32)
             + b4_ref[...])
    gate = jax.nn.sigmoid(gate1 + gate2)
    o1_ref[...] = out + gate * (t - out)
    o2_ref[...] = t + gate * (out - t)


def _post(partials, sup, x, r, w2, w3, w4, b1, b2, b3, b4, eps):
    row = pl.BlockSpec((BLK, D), lambda i: (i, 0))
    par0 = pl.BlockSpec((1, BLK, D), lambda i: (0, i, 0))
    par1 = pl.BlockSpec((1, BLK, D), lambda i: (1, i, 0))
    mat = pl.BlockSpec((D, D), lambda i: (0, 0))
    vec = pl.BlockSpec((1, D), lambda i: (0, 0))
    sca = pl.BlockSpec(memory_space=pltpu.SMEM)
    out = jax.ShapeDtypeStruct((N, D), jnp.float32)
    return pl.pallas_call(
        _post_body,
        grid=(N // BLK,),
        in_specs=[par0, par1, row, row, row, mat, mat, mat,
                  vec, vec, vec, vec, sca],
        out_specs=[row, row],
        out_shape=[out, out],
    )(partials, partials, sup, x, r, w2, w3, w4,
      b1.reshape(1, D), b2.reshape(1, D), b3.reshape(1, D),
      b4.reshape(1, D), eps)


def kernel(input, res_input, edge_index, w1, w2, w3, w4, epsilo, b1, b2, b3, b4):
    src = edge_index[0].astype(jnp.int32)
    dst = edge_index[1].astype(jnp.int32)
    zeros = jnp.zeros((N, D), jnp.float32)

    support = _pre_sup(input, w1)
    partials = _sc_agg(support, src, dst, zeros)
    return _post(partials, support, input, res_input,
                 w2, w3, w4, b1, b2, b3, b4, epsilo)
